# Initial kernel scaffold; baseline (speedup 1.0000x reference)
import jax
import jax.numpy as jnp
from jax.experimental import pallas as pl


def kernel(x, params):
    raise NotImplementedError("write your pallas kernel here")


# speedup vs baseline: 95.8384x; 95.8384x over previous
import functools

import jax
import jax.numpy as jnp
from jax.experimental import pallas as pl

N = 10
BB = 128


def _ln(x, g, b):
    m = x.mean(-1, keepdims=True)
    v = jnp.mean((x - m) ** 2, axis=-1, keepdims=True)
    return (x - m) / jnp.sqrt(v + 1e-5) * g + b


def _elu(x):
    return jnp.where(x > 0, x, jnp.exp(x) - 1.0)


def _leaky(x):
    return jnp.where(x > 0, x, 0.2 * x)


def _dot(a, b):
    return jax.lax.dot_general(a, b, (((1,), (0,)), ((), ())),
                               preferred_element_type=jnp.float32)


def _attention(e, mult, has):
    masked = jnp.where(has, e, -jnp.inf)
    emax = masked.max(axis=1, keepdims=True)
    emax = jnp.where(jnp.isneginf(emax), 0.0, emax)
    ee = mult * jnp.exp(e - emax)
    den = ee.sum(axis=1, keepdims=True)
    return ee / (den + 1e-16)


def _body(topo_ref, nf_ref, routing_ref, traffic_ref,
          we_ref, be_ref, gne_ref, bne_ref,
          wg1_ref, as1_ref, ad1_ref, bg1_ref, g1_ref, b1_ref,
          wg2_ref, as2_ref, ad2_ref, bg2_ref, g2_ref, b2_ref,
          wp_ref, bp_ref,
          wr_ref, br_ref, gr_ref, brb_ref,
          wt_ref, bt_ref, gt_ref, btb_ref,
          wf_ref, bf_ref, gf_ref, bfb_ref,
          wa1_ref, ba1_ref, wa2_ref, ba2_ref, wa3_ref, ba3_ref,
          out_ref):
    topo = topo_ref[...]
    nf = nf_ref[...]

    h2d = jax.nn.relu(_dot(nf.reshape(BB * N, 4), we_ref[...]) + be_ref[...])
    h2d = _ln(h2d, gne_ref[...], bne_ref[...])

    nz = (topo != 0.0).astype(jnp.float32)
    nnz = nz.sum(axis=(1, 2), keepdims=True)
    ii = jax.lax.broadcasted_iota(jnp.int32, (BB, N, N), 1)
    jj = jax.lax.broadcasted_iota(jnp.int32, (BB, N, N), 2)
    at00 = jnp.logical_and(ii == 0, jj == 0)
    mult = nz + jnp.where(at00, 100.0 - nnz, 0.0)
    has = mult > 0.0

    identity = _dot(h2d, wp_ref[...]) + bp_ref[...]

    hh = _dot(h2d, wg1_ref[...])
    as1 = as1_ref[...]
    ad1 = ad1_ref[...]
    outs = []
    for hd in range(4):
        hh_h = hh[:, hd * 64:(hd + 1) * 64]
        es = (hh_h * as1[hd:hd + 1, :]).sum(-1, keepdims=True)
        ed = (hh_h * ad1[hd:hd + 1, :]).sum(-1, keepdims=True)
        es3 = es.reshape(BB, N, 1)
        ed3 = ed.reshape(BB, N, 1)
        e = _leaky(es3 + jnp.swapaxes(ed3, 1, 2))
        alpha = _attention(e, mult, has)
        hh3 = hh_h.reshape(BB, N, 64)
        acc = jnp.zeros((BB, N, 64), jnp.float32)
        for s in range(N):
            acc = acc + alpha[:, s, :, None] * hh3[:, s, None, :]
        outs.append(acc)
    x1 = jnp.concatenate(outs, axis=-1).reshape(BB * N, 256) + bg1_ref[...]
    x1 = _elu(_ln(x1, g1_ref[...], b1_ref[...]))

    hh2 = _dot(x1, wg2_ref[...])
    es = (hh2 * as2_ref[...]).sum(-1, keepdims=True)
    ed = (hh2 * ad2_ref[...]).sum(-1, keepdims=True)
    es3 = es.reshape(BB, N, 1)
    ed3 = ed.reshape(BB, N, 1)
    e = _leaky(es3 + jnp.swapaxes(ed3, 1, 2))
    alpha = _attention(e, mult, has)
    hh23 = hh2.reshape(BB, N, 64)
    acc = jnp.zeros((BB, N, 64), jnp.float32)
    for s in range(N):
        acc = acc + alpha[:, s, :, None] * hh23[:, s, None, :]
    x2 = acc.reshape(BB * N, 64) + bg2_ref[...]
    x2 = _ln(x2, g2_ref[...], b2_ref[...])

    out = _elu(x2 + identity)
    g = out.reshape(BB, N, 64).mean(axis=1)

    r = _ln(jax.nn.relu(_dot(routing_ref[...], wr_ref[...]) + br_ref[...]),
            gr_ref[...], brb_ref[...])
    t = _ln(jax.nn.relu(_dot(traffic_ref[...], wt_ref[...]) + bt_ref[...]),
            gt_ref[...], btb_ref[...])

    comb = jnp.concatenate([g, r, t], axis=1)
    feat = _ln(jax.nn.relu(_dot(comb, wf_ref[...]) + bf_ref[...]),
               gf_ref[...], bfb_ref[...])
    h1 = jax.nn.relu(_dot(feat, wa1_ref[...]) + ba1_ref[...])
    hh2f = jax.nn.relu(_dot(h1, wa2_ref[...]) + ba2_ref[...])
    out_ref[...] = _dot(hh2f, wa3_ref[...]) + ba3_ref[...]


@jax.jit
def kernel(x, params):
    p = params
    B = x.shape[0]
    topo = x[:, 0:100].reshape(B, N, N)
    nf = jnp.concatenate(
        [x[:, 100:130].reshape(B, N, 3), x[:, 245:255][..., None]], axis=-1)
    routing = x[:, 130:140]
    traffic = x[:, 240:245]

    def v(name):
        a = p[name]
        return a.reshape(1, -1) if a.ndim == 1 else a

    param_names = ['we', 'be', 'gne', 'bne',
                   'wg1', 'as1', 'ad1', 'bg1', 'g1', 'b1',
                   'wg2', 'as2', 'ad2', 'bg2', 'g2', 'b2',
                   'wp', 'bp',
                   'wr', 'br', 'gr', 'brb',
                   'wt', 'bt', 'gt', 'btb',
                   'wf', 'bf', 'gf', 'bfb',
                   'wa1', 'ba1', 'wa2', 'ba2', 'wa3', 'ba3']
    pvals = [v(n) for n in param_names]

    def const_spec(a):
        nd = a.ndim
        return pl.BlockSpec(a.shape, lambda i, _nd=nd: (0,) * _nd)

    in_specs = [
        pl.BlockSpec((BB, N, N), lambda i: (i, 0, 0)),
        pl.BlockSpec((BB, N, 4), lambda i: (i, 0, 0)),
        pl.BlockSpec((BB, 10), lambda i: (i, 0)),
        pl.BlockSpec((BB, 5), lambda i: (i, 0)),
    ] + [const_spec(a) for a in pvals]

    out = pl.pallas_call(
        _body,
        grid=(B // BB,),
        in_specs=in_specs,
        out_specs=pl.BlockSpec((BB, 10), lambda i: (i, 0)),
        out_shape=jax.ShapeDtypeStruct((B, 10), jnp.float32),
    )(topo, nf, routing, traffic, *pvals)
    return out


# speedup vs baseline: 111.6455x; 1.1649x over previous
import functools

import jax
import jax.numpy as jnp
from jax.experimental import pallas as pl

N = 10
BB = 256


def _ln(x, g, b):
    m = x.mean(-1, keepdims=True)
    v = jnp.mean((x - m) ** 2, axis=-1, keepdims=True)
    return (x - m) / jnp.sqrt(v + 1e-5) * g + b


def _elu(x):
    return jnp.where(x > 0, x, jnp.exp(x) - 1.0)


def _leaky(x):
    return jnp.where(x > 0, x, 0.2 * x)


def _dot(a, b):
    return jax.lax.dot_general(a, b, (((1,), (0,)), ((), ())),
                               preferred_element_type=jnp.float32)


def _attention(e, mult, has):
    masked = jnp.where(has, e, -jnp.inf)
    emax = masked.max(axis=1, keepdims=True)
    emax = jnp.where(jnp.isneginf(emax), 0.0, emax)
    ee = mult * jnp.exp(e - emax)
    den = ee.sum(axis=1, keepdims=True)
    return ee / (den + 1e-16)


def _body(topo_ref, nf_ref, routing_ref, traffic_ref,
          we_ref, be_ref, gne_ref, bne_ref,
          wg1_ref, as1_ref, ad1_ref, bg1_ref, g1_ref, b1_ref,
          wg2_ref, as2_ref, ad2_ref, bg2_ref, g2_ref, b2_ref,
          wp_ref, bp_ref,
          wr_ref, br_ref, gr_ref, brb_ref,
          wt_ref, bt_ref, gt_ref, btb_ref,
          wf_ref, bf_ref, gf_ref, bfb_ref,
          wa1_ref, ba1_ref, wa2_ref, ba2_ref, wa3_ref, ba3_ref,
          out_ref):
    topo = topo_ref[...]
    nf = nf_ref[...]

    h2d = jax.nn.relu(_dot(nf.reshape(BB * N, 4), we_ref[...]) + be_ref[...])
    h2d = _ln(h2d, gne_ref[...], bne_ref[...])

    nz = (topo != 0.0).astype(jnp.float32)
    nnz = nz.sum(axis=(1, 2), keepdims=True)
    ii = jax.lax.broadcasted_iota(jnp.int32, (BB, N, N), 1)
    jj = jax.lax.broadcasted_iota(jnp.int32, (BB, N, N), 2)
    at00 = jnp.logical_and(ii == 0, jj == 0)
    mult = nz + jnp.where(at00, 100.0 - nnz, 0.0)
    has = mult > 0.0

    identity = _dot(h2d, wp_ref[...]) + bp_ref[...]

    hh = _dot(h2d, wg1_ref[...])
    as1 = as1_ref[...]
    ad1 = ad1_ref[...]
    outs = []
    for hd in range(4):
        hh_h = hh[:, hd * 64:(hd + 1) * 64]
        es = (hh_h * as1[hd:hd + 1, :]).sum(-1, keepdims=True)
        ed = (hh_h * ad1[hd:hd + 1, :]).sum(-1, keepdims=True)
        es3 = es.reshape(BB, N, 1)
        ed3 = ed.reshape(BB, N, 1)
        e = _leaky(es3 + jnp.swapaxes(ed3, 1, 2))
        alpha = _attention(e, mult, has)
        hh3 = hh_h.reshape(BB, N, 64)
        acc = jax.lax.dot_general(
            alpha, hh3, (((1,), (1,)), ((0,), (0,))),
            preferred_element_type=jnp.float32)
        outs.append(acc)
    x1 = jnp.concatenate(outs, axis=-1).reshape(BB * N, 256) + bg1_ref[...]
    x1 = _elu(_ln(x1, g1_ref[...], b1_ref[...]))

    hh2 = _dot(x1, wg2_ref[...])
    es = (hh2 * as2_ref[...]).sum(-1, keepdims=True)
    ed = (hh2 * ad2_ref[...]).sum(-1, keepdims=True)
    es3 = es.reshape(BB, N, 1)
    ed3 = ed.reshape(BB, N, 1)
    e = _leaky(es3 + jnp.swapaxes(ed3, 1, 2))
    alpha = _attention(e, mult, has)
    hh23 = hh2.reshape(BB, N, 64)
    acc = jax.lax.dot_general(
        alpha, hh23, (((1,), (1,)), ((0,), (0,))),
        preferred_element_type=jnp.float32)
    x2 = acc.reshape(BB * N, 64) + bg2_ref[...]
    x2 = _ln(x2, g2_ref[...], b2_ref[...])

    out = _elu(x2 + identity)
    g = out.reshape(BB, N, 64).mean(axis=1)

    r = _ln(jax.nn.relu(_dot(routing_ref[...], wr_ref[...]) + br_ref[...]),
            gr_ref[...], brb_ref[...])
    t = _ln(jax.nn.relu(_dot(traffic_ref[...], wt_ref[...]) + bt_ref[...]),
            gt_ref[...], btb_ref[...])

    comb = jnp.concatenate([g, r, t], axis=1)
    feat = _ln(jax.nn.relu(_dot(comb, wf_ref[...]) + bf_ref[...]),
               gf_ref[...], bfb_ref[...])
    h1 = jax.nn.relu(_dot(feat, wa1_ref[...]) + ba1_ref[...])
    hh2f = jax.nn.relu(_dot(h1, wa2_ref[...]) + ba2_ref[...])
    out_ref[...] = _dot(hh2f, wa3_ref[...]) + ba3_ref[...]


@jax.jit
def kernel(x, params):
    p = params
    B = x.shape[0]
    topo = x[:, 0:100].reshape(B, N, N)
    nf = jnp.concatenate(
        [x[:, 100:130].reshape(B, N, 3), x[:, 245:255][..., None]], axis=-1)
    routing = x[:, 130:140]
    traffic = x[:, 240:245]

    def v(name):
        a = p[name]
        return a.reshape(1, -1) if a.ndim == 1 else a

    param_names = ['we', 'be', 'gne', 'bne',
                   'wg1', 'as1', 'ad1', 'bg1', 'g1', 'b1',
                   'wg2', 'as2', 'ad2', 'bg2', 'g2', 'b2',
                   'wp', 'bp',
                   'wr', 'br', 'gr', 'brb',
                   'wt', 'bt', 'gt', 'btb',
                   'wf', 'bf', 'gf', 'bfb',
                   'wa1', 'ba1', 'wa2', 'ba2', 'wa3', 'ba3']
    pvals = [v(n) for n in param_names]

    def const_spec(a):
        nd = a.ndim
        return pl.BlockSpec(a.shape, lambda i, _nd=nd: (0,) * _nd)

    in_specs = [
        pl.BlockSpec((BB, N, N), lambda i: (i, 0, 0)),
        pl.BlockSpec((BB, N, 4), lambda i: (i, 0, 0)),
        pl.BlockSpec((BB, 10), lambda i: (i, 0)),
        pl.BlockSpec((BB, 5), lambda i: (i, 0)),
    ] + [const_spec(a) for a in pvals]

    out = pl.pallas_call(
        _body,
        grid=(B // BB,),
        in_specs=in_specs,
        out_specs=pl.BlockSpec((BB, 10), lambda i: (i, 0)),
        out_shape=jax.ShapeDtypeStruct((B, 10), jnp.float32),
    )(topo, nf, routing, traffic, *pvals)
    return out


# speedup vs baseline: 133.2703x; 1.1937x over previous
import functools

import jax
import jax.numpy as jnp
from jax.experimental import pallas as pl

N = 10
BB = 256


def _ln(x, g, b):
    m = x.mean(-1, keepdims=True)
    v = jnp.mean((x - m) ** 2, axis=-1, keepdims=True)
    return (x - m) / jnp.sqrt(v + 1e-5) * g + b


def _elu(x):
    return jnp.where(x > 0, x, jnp.exp(x) - 1.0)


def _leaky(x):
    return jnp.where(x > 0, x, 0.2 * x)


def _dot(a, b):
    return jax.lax.dot_general(a, b, (((1,), (0,)), ((), ())),
                               preferred_element_type=jnp.float32)


def _attention(e, mult, has):
    masked = jnp.where(has, e, -jnp.inf)
    emax = masked.max(axis=1, keepdims=True)
    emax = jnp.where(jnp.isneginf(emax), 0.0, emax)
    ee = mult * jnp.exp(e - emax)
    den = ee.sum(axis=1, keepdims=True)
    return ee / (den + 1e-16)


def _body(topo_ref, nf_ref, routing_ref, traffic_ref,
          we_ref, be_ref, gne_ref, bne_ref,
          wg1_ref, as1_ref, ad1_ref, bg1_ref, g1_ref, b1_ref,
          wg2_ref, as2_ref, ad2_ref, bg2_ref, g2_ref, b2_ref,
          wp_ref, bp_ref,
          wr_ref, br_ref, gr_ref, brb_ref,
          wt_ref, bt_ref, gt_ref, btb_ref,
          wf_ref, bf_ref, gf_ref, bfb_ref,
          wa1_ref, ba1_ref, wa2_ref, ba2_ref, wa3_ref, ba3_ref,
          out_ref):
    topo = topo_ref[...]
    nf = nf_ref[...]

    h2d = jax.nn.relu(_dot(nf.reshape(BB * N, 4), we_ref[...]) + be_ref[...])
    h2d = _ln(h2d, gne_ref[...], bne_ref[...])

    nz = (topo != 0.0).astype(jnp.float32)
    nnz = nz.sum(axis=(1, 2), keepdims=True)
    ii = jax.lax.broadcasted_iota(jnp.int32, (BB, N, N), 1)
    jj = jax.lax.broadcasted_iota(jnp.int32, (BB, N, N), 2)
    at00 = jnp.logical_and(ii == 0, jj == 0)
    mult = nz + jnp.where(at00, 100.0 - nnz, 0.0)
    has = mult > 0.0

    identity = _dot(h2d, wp_ref[...]) + bp_ref[...]

    hh = _dot(h2d, wg1_ref[...])
    as1 = as1_ref[...]
    ad1 = ad1_ref[...]
    outs = []
    for hd in range(4):
        hh_h = hh[:, hd * 64:(hd + 1) * 64]
        hh3 = hh_h.reshape(BB, N, 64)
        es3 = jax.lax.dot_general(
            hh3, as1[hd:hd + 1, :].T, (((2,), (0,)), ((), ())),
            preferred_element_type=jnp.float32)
        ed_l = jax.lax.dot_general(
            ad1[hd:hd + 1, :], hh3, (((1,), (2,)), ((), ())),
            preferred_element_type=jnp.float32)
        e = _leaky(es3 + ed_l.reshape(BB, 1, N))
        alpha = _attention(e, mult, has)
        acc = jax.lax.dot_general(
            alpha, hh3, (((1,), (1,)), ((0,), (0,))),
            preferred_element_type=jnp.float32)
        outs.append(acc)
    x1 = jnp.concatenate(outs, axis=-1).reshape(BB * N, 256) + bg1_ref[...]
    x1 = _elu(_ln(x1, g1_ref[...], b1_ref[...]))

    hh2 = _dot(x1, wg2_ref[...])
    hh23 = hh2.reshape(BB, N, 64)
    es3 = jax.lax.dot_general(
        hh23, as2_ref[...].T, (((2,), (0,)), ((), ())),
        preferred_element_type=jnp.float32)
    ed_l = jax.lax.dot_general(
        ad2_ref[...], hh23, (((1,), (2,)), ((), ())),
        preferred_element_type=jnp.float32)
    e = _leaky(es3 + ed_l.reshape(BB, 1, N))
    alpha = _attention(e, mult, has)
    acc = jax.lax.dot_general(
        alpha, hh23, (((1,), (1,)), ((0,), (0,))),
        preferred_element_type=jnp.float32)
    x2 = acc.reshape(BB * N, 64) + bg2_ref[...]
    x2 = _ln(x2, g2_ref[...], b2_ref[...])

    out = _elu(x2 + identity)
    g = out.reshape(BB, N, 64).mean(axis=1)

    r = _ln(jax.nn.relu(_dot(routing_ref[...], wr_ref[...]) + br_ref[...]),
            gr_ref[...], brb_ref[...])
    t = _ln(jax.nn.relu(_dot(traffic_ref[...], wt_ref[...]) + bt_ref[...]),
            gt_ref[...], btb_ref[...])

    comb = jnp.concatenate([g, r, t], axis=1)
    feat = _ln(jax.nn.relu(_dot(comb, wf_ref[...]) + bf_ref[...]),
               gf_ref[...], bfb_ref[...])
    h1 = jax.nn.relu(_dot(feat, wa1_ref[...]) + ba1_ref[...])
    hh2f = jax.nn.relu(_dot(h1, wa2_ref[...]) + ba2_ref[...])
    out_ref[...] = _dot(hh2f, wa3_ref[...]) + ba3_ref[...]


@jax.jit
def kernel(x, params):
    p = params
    B = x.shape[0]
    topo = x[:, 0:100].reshape(B, N, N)
    nf = jnp.concatenate(
        [x[:, 100:130].reshape(B, N, 3), x[:, 245:255][..., None]], axis=-1)
    routing = x[:, 130:140]
    traffic = x[:, 240:245]

    def v(name):
        a = p[name]
        return a.reshape(1, -1) if a.ndim == 1 else a

    param_names = ['we', 'be', 'gne', 'bne',
                   'wg1', 'as1', 'ad1', 'bg1', 'g1', 'b1',
                   'wg2', 'as2', 'ad2', 'bg2', 'g2', 'b2',
                   'wp', 'bp',
                   'wr', 'br', 'gr', 'brb',
                   'wt', 'bt', 'gt', 'btb',
                   'wf', 'bf', 'gf', 'bfb',
                   'wa1', 'ba1', 'wa2', 'ba2', 'wa3', 'ba3']
    pvals = [v(n) for n in param_names]

    def const_spec(a):
        nd = a.ndim
        return pl.BlockSpec(a.shape, lambda i, _nd=nd: (0,) * _nd)

    in_specs = [
        pl.BlockSpec((BB, N, N), lambda i: (i, 0, 0)),
        pl.BlockSpec((BB, N, 4), lambda i: (i, 0, 0)),
        pl.BlockSpec((BB, 10), lambda i: (i, 0)),
        pl.BlockSpec((BB, 5), lambda i: (i, 0)),
    ] + [const_spec(a) for a in pvals]

    out = pl.pallas_call(
        _body,
        grid=(B // BB,),
        in_specs=in_specs,
        out_specs=pl.BlockSpec((BB, 10), lambda i: (i, 0)),
        out_shape=jax.ShapeDtypeStruct((B, 10), jnp.float32),
    )(topo, nf, routing, traffic, *pvals)
    return out


# speedup vs baseline: 212.1862x; 1.5921x over previous
import numpy as np

import jax
import jax.numpy as jnp
from jax.experimental import pallas as pl

N = 10
BB = 256


def _elu(x):
    return jnp.where(x > 0, x, jnp.exp(x) - 1.0)


def _leaky(x):
    return jnp.where(x > 0, x, 0.2 * x)


def _dot(a, b):
    return jax.lax.dot_general(a, b, (((1,), (0,)), ((), ())),
                               preferred_element_type=jnp.float32)


def _ln(x, g, b, om):
    m = _dot(x, om)
    d = x - m
    v = _dot(d * d, om)
    return d * jax.lax.rsqrt(v + 1e-5) * g + b


def _row_scores(a_row, hh3):
    r = jax.lax.dot_general(a_row, hh3, (((1,), (2,)), ((), ())),
                            preferred_element_type=jnp.float32)
    return r.reshape(hh3.shape[0], N)


def _gat_attention(hh3, a_src, a_dst, mult, p_ref, ssum_ref):
    es = _row_scores(a_src, hh3)
    ed = _row_scores(a_dst, hh3)
    esd = jnp.concatenate([es, ed], axis=-1)
    e = _leaky(_dot(esd, p_ref[...]))
    c = e.max(axis=-1, keepdims=True)
    ee = mult * jnp.exp(e - c)
    den = _dot(ee, ssum_ref[...])
    r = 1.0 / (den + 1e-16)
    r100 = jax.lax.dot_general(r, ssum_ref[...], (((1,), (1,)), ((), ())),
                               preferred_element_type=jnp.float32)
    alpha = ee * r100
    alpha3 = alpha.reshape(hh3.shape[0], N, N)
    return jax.lax.dot_general(
        alpha3, hh3, (((1,), (1,)), ((0,), (0,))),
        preferred_element_type=jnp.float32)


def _body(xb_ref, nf_ref,
          p_ref, ssum_ref, o100_ref, om32_ref, om64_ref, om256_ref, on10_ref,
          we_ref, be_ref, gne_ref, bne_ref,
          wg1_ref, as1_ref, ad1_ref, bg1_ref, g1_ref, b1_ref,
          wg2_ref, as2_ref, ad2_ref, bg2_ref, g2_ref, b2_ref,
          wp_ref, bp_ref,
          wr_ref, br_ref, gr_ref, brb_ref,
          wt_ref, bt_ref, gt_ref, btb_ref,
          wf_ref, bf_ref, gf_ref, bfb_ref,
          wa1_ref, ba1_ref, wa2_ref, ba2_ref, wa3_ref, ba3_ref,
          out_ref):
    xb = xb_ref[...]
    nf = nf_ref[...]

    h2d = jax.nn.relu(_dot(nf.reshape(BB * N, 4), we_ref[...]) + be_ref[...])
    h2d = _ln(h2d, gne_ref[...], bne_ref[...], om32_ref[...])

    tflat = xb[:, 0:100]
    nz = (tflat != 0.0).astype(jnp.float32)
    nnz = _dot(nz, o100_ref[...])
    lane = jax.lax.broadcasted_iota(jnp.int32, (BB, 100), 1)
    mult = nz + jnp.where(lane == 0, 100.0 - nnz, 0.0)

    identity = _dot(h2d, wp_ref[...]) + bp_ref[...]

    hh = _dot(h2d, wg1_ref[...])
    as1 = as1_ref[...]
    ad1 = ad1_ref[...]
    outs = []
    for hd in range(4):
        hh3 = hh[:, hd * 64:(hd + 1) * 64].reshape(BB, N, 64)
        acc = _gat_attention(hh3, as1[hd:hd + 1, :], ad1[hd:hd + 1, :],
                             mult, p_ref, ssum_ref)
        outs.append(acc)
    x1 = jnp.concatenate(outs, axis=-1).reshape(BB * N, 256) + bg1_ref[...]
    x1 = _elu(_ln(x1, g1_ref[...], b1_ref[...], om256_ref[...]))

    hh23 = _dot(x1, wg2_ref[...]).reshape(BB, N, 64)
    acc = _gat_attention(hh23, as2_ref[...], ad2_ref[...],
                         mult, p_ref, ssum_ref)
    x2 = acc.reshape(BB * N, 64) + bg2_ref[...]
    x2 = _ln(x2, g2_ref[...], b2_ref[...], om64_ref[...])

    out = _elu(x2 + identity)
    g = jax.lax.dot_general(on10_ref[...], out.reshape(BB, N, 64),
                            (((1,), (1,)), ((), ())),
                            preferred_element_type=jnp.float32).reshape(BB, 64)

    r = _ln(jax.nn.relu(_dot(xb[:, 130:140], wr_ref[...]) + br_ref[...]),
            gr_ref[...], brb_ref[...], om64_ref[...])
    t = _ln(jax.nn.relu(_dot(xb[:, 240:245], wt_ref[...]) + bt_ref[...]),
            gt_ref[...], btb_ref[...], om32_ref[...])

    comb = jnp.concatenate([g, r, t], axis=1)
    feat = _ln(jax.nn.relu(_dot(comb, wf_ref[...]) + bf_ref[...]),
               gf_ref[...], bfb_ref[...], om256_ref[...])
    h1 = jax.nn.relu(_dot(feat, wa1_ref[...]) + ba1_ref[...])
    h2f = jax.nn.relu(_dot(h1, wa2_ref[...]) + ba2_ref[...])
    out_ref[...] = _dot(h2f, wa3_ref[...]) + ba3_ref[...]


def _pattern_consts():
    P = np.zeros((2 * N, N * N), np.float32)
    S = np.zeros((N * N, N), np.float32)
    for s in range(N):
        for d in range(N):
            P[s, s * N + d] = 1.0
            P[N + d, s * N + d] = 1.0
            S[s * N + d, d] = 1.0
    return jnp.asarray(P), jnp.asarray(S)


@jax.jit
def kernel(x, params):
    p = params
    B = x.shape[0]
    nf = jnp.concatenate(
        [x[:, 100:130].reshape(B, N, 3), x[:, 245:255][..., None]], axis=-1)

    P, S = _pattern_consts()
    o100 = jnp.ones((100, 1), jnp.float32)
    om32 = jnp.full((32, 1), 1.0 / 32, jnp.float32)
    om64 = jnp.full((64, 1), 1.0 / 64, jnp.float32)
    om256 = jnp.full((256, 1), 1.0 / 256, jnp.float32)
    on10 = jnp.full((1, N), 1.0 / N, jnp.float32)
    consts = [P, S, o100, om32, om64, om256, on10]

    def v(name):
        a = p[name]
        return a.reshape(1, -1) if a.ndim == 1 else a

    param_names = ['we', 'be', 'gne', 'bne',
                   'wg1', 'as1', 'ad1', 'bg1', 'g1', 'b1',
                   'wg2', 'as2', 'ad2', 'bg2', 'g2', 'b2',
                   'wp', 'bp',
                   'wr', 'br', 'gr', 'brb',
                   'wt', 'bt', 'gt', 'btb',
                   'wf', 'bf', 'gf', 'bfb',
                   'wa1', 'ba1', 'wa2', 'ba2', 'wa3', 'ba3']
    pvals = consts + [v(n) for n in param_names]

    def const_spec(a):
        nd = a.ndim
        return pl.BlockSpec(a.shape, lambda i, _nd=nd: (0,) * _nd)

    in_specs = [
        pl.BlockSpec((BB, 255), lambda i: (i, 0)),
        pl.BlockSpec((BB, N, 4), lambda i: (i, 0, 0)),
    ] + [const_spec(a) for a in pvals]

    out = pl.pallas_call(
        _body,
        grid=(B // BB,),
        in_specs=in_specs,
        out_specs=pl.BlockSpec((BB, 10), lambda i: (i, 0)),
        out_shape=jax.ShapeDtypeStruct((B, 10), jnp.float32),
    )(x, nf, *pvals)
    return out
